# baseline (speedup 1.0000x reference)
import functools

import jax
import jax.numpy as jnp
from jax import lax
from jax.experimental import pallas as pl
from jax.experimental.pallas import tpu as pltpu
from jax.experimental.pallas import tpu_sc as plsc

L = 100000
H = 100000
D = 120
Q = 1024

TILE_L = 2000
N_TILES = L // TILE_L


def _argmax_body(l_ref, s_ref, out_ref, bv_ref, bi_ref):
    i = pl.program_id(0)

    @pl.when(i == 0)
    def _init():
        bv_ref[...] = jnp.full((1, Q), -jnp.inf, dtype=jnp.float32)
        bi_ref[...] = jnp.zeros((1, Q), dtype=jnp.int32)

    scores = jnp.dot(l_ref[...], s_ref[...],
                     preferred_element_type=jnp.float32)
    tmax = jnp.max(scores, axis=0, keepdims=True)
    rows = lax.broadcasted_iota(jnp.int32, (TILE_L, Q), 0)
    cand = jnp.where(scores == tmax, rows, TILE_L)
    targ = jnp.min(cand, axis=0, keepdims=True) + i * TILE_L

    upd = tmax > bv_ref[...]
    bv_ref[...] = jnp.where(upd, tmax, bv_ref[...])
    bi_ref[...] = jnp.where(upd, targ, bi_ref[...])

    @pl.when(i == pl.num_programs(0) - 1)
    def _fin():
        out_ref[...] = bi_ref[...]


def _argmax_call(lDict, sample):
    return pl.pallas_call(
        _argmax_body,
        grid=(N_TILES,),
        in_specs=[
            pl.BlockSpec((TILE_L, D), lambda i: (i, 0)),
            pl.BlockSpec((D, Q), lambda i: (0, 0)),
        ],
        out_specs=pl.BlockSpec((1, Q), lambda i: (0, 0)),
        out_shape=jax.ShapeDtypeStruct((1, Q), jnp.int32),
        scratch_shapes=[
            pltpu.VMEM((1, Q), jnp.float32),
            pltpu.VMEM((1, Q), jnp.int32),
        ],
    )(lDict, sample)


DPAD = 128


def _make_gather():
    info = plsc.get_sparse_core_info()
    nw = info.num_cores * info.num_subcores
    b_per_w = Q // nw
    mesh = plsc.VectorSubcoreMesh(core_axis_name="c", subcore_axis_name="s")

    @functools.partial(
        pl.kernel,
        mesh=mesh,
        out_type=jax.ShapeDtypeStruct((Q, DPAD), jnp.float32),
        scratch_types=[
            pltpu.VMEM((b_per_w,), jnp.int32),
            pltpu.VMEM((b_per_w, DPAD), jnp.float32),
            pltpu.SemaphoreType.DMA,
        ],
    )
    def gather(table_hbm, idx_hbm, out_hbm, idx_v, rows_v, sem):
        wid = lax.axis_index("s") * info.num_cores + lax.axis_index("c")
        base = wid * b_per_w
        pltpu.sync_copy(idx_hbm.at[pl.ds(base, b_per_w)], idx_v)
        pltpu.async_copy(table_hbm.at[idx_v], rows_v, sem).wait()
        pltpu.sync_copy(rows_v, out_hbm.at[pl.ds(base, b_per_w)])

    return gather


@functools.lru_cache(maxsize=1)
def _gather_call():
    return _make_gather()


def kernel(sample, lDict, hDict):
    idx = _argmax_call(lDict, sample)
    hPad = jnp.pad(hDict, ((0, 0), (0, DPAD - D)))
    outPad = _gather_call()(hPad, idx.reshape(Q))
    return outPad[:, :D]


# speedup vs baseline: 1.0843x; 1.0843x over previous
import functools

import jax
import jax.numpy as jnp
from jax import lax
from jax.experimental import pallas as pl
from jax.experimental.pallas import tpu as pltpu
from jax.experimental.pallas import tpu_sc as plsc

L = 100000
H = 100000
D = 120
Q = 1024

TILE_L = 2000
N_TILES = L // TILE_L


def _argmax_body(l_ref, s_ref, out_ref, bv_ref, bi_ref):
    i = pl.program_id(0)

    @pl.when(i == 0)
    def _init():
        bv_ref[...] = jnp.full((1, Q), -jnp.inf, dtype=jnp.float32)
        bi_ref[...] = jnp.zeros((1, Q), dtype=jnp.int32)

    scores = jnp.dot(l_ref[...], s_ref[...],
                     preferred_element_type=jnp.float32)
    tmax = jnp.max(scores, axis=0, keepdims=True)
    rows = lax.broadcasted_iota(jnp.int32, (TILE_L, Q), 0)
    cand = jnp.where(scores == tmax, rows, TILE_L)
    targ = jnp.min(cand, axis=0, keepdims=True) + i * TILE_L

    upd = tmax > bv_ref[...]
    bv_ref[...] = jnp.where(upd, tmax, bv_ref[...])
    bi_ref[...] = jnp.where(upd, targ, bi_ref[...])

    @pl.when(i == pl.num_programs(0) - 1)
    def _fin():
        out_ref[...] = bi_ref[...]


def _argmax_call(lDict, sample):
    return pl.pallas_call(
        _argmax_body,
        grid=(N_TILES,),
        in_specs=[
            pl.BlockSpec((TILE_L, D), lambda i: (i, 0)),
            pl.BlockSpec((D, Q), lambda i: (0, 0)),
        ],
        out_specs=pl.BlockSpec((1, Q), lambda i: (0, 0)),
        out_shape=jax.ShapeDtypeStruct((1, Q), jnp.int32),
        scratch_shapes=[
            pltpu.VMEM((1, Q), jnp.float32),
            pltpu.VMEM((1, Q), jnp.int32),
        ],
    )(lDict, sample)


def _make_gather():
    info = plsc.get_sparse_core_info()
    nw = info.num_cores * info.num_subcores
    b_per_w = Q // nw
    mesh = plsc.VectorSubcoreMesh(core_axis_name="c", subcore_axis_name="s")

    @functools.partial(
        pl.kernel,
        mesh=mesh,
        out_type=jax.ShapeDtypeStruct((Q, D), jnp.float32),
        scratch_types=[
            pltpu.VMEM((b_per_w,), jnp.int32),
            pltpu.VMEM((b_per_w, D), jnp.float32),
            pltpu.SemaphoreType.DMA,
        ],
    )
    def gather(table_hbm, idx_hbm, out_hbm, idx_v, rows_v, sem):
        wid = lax.axis_index("s") * info.num_cores + lax.axis_index("c")
        base = wid * b_per_w
        pltpu.sync_copy(idx_hbm.at[pl.ds(base, b_per_w)], idx_v)
        handles = []
        for c in range(b_per_w // 16):
            vec = idx_v[pl.ds(c * 16, 16)]
            for j in range(16):
                handles.append(pltpu.async_copy(
                    table_hbm.at[vec[j]], rows_v.at[c * 16 + j], sem))
        for h in handles:
            h.wait()
        pltpu.sync_copy(rows_v, out_hbm.at[pl.ds(base, b_per_w)])

    return gather


@functools.lru_cache(maxsize=1)
def _gather_call():
    return _make_gather()


def kernel(sample, lDict, hDict):
    idx = _argmax_call(lDict, sample)
    return _gather_call()(hDict, idx.reshape(Q))


# speedup vs baseline: 1.4239x; 1.3132x over previous
import functools

import jax
import jax.numpy as jnp
from jax import lax
from jax.experimental import pallas as pl
from jax.experimental.pallas import tpu as pltpu
from jax.experimental.pallas import tpu_sc as plsc

L = 100000
H = 100000
D = 120
Q = 1024

TILE_L = 2048
N_TILES = pl.cdiv(L, TILE_L)


def _argmax_body(lt_ref, ht_ref, s_ref, idx_ref, hrm_ref, bv_ref, bi_ref):
    i = pl.program_id(0)

    @pl.when(i == 0)
    def _init():
        bv_ref[...] = jnp.full((1, Q), -jnp.inf, dtype=jnp.float32)
        bi_ref[...] = jnp.zeros((1, Q), dtype=jnp.int32)

    di = lax.broadcasted_iota(jnp.int32, (D, D), 0)
    dj = lax.broadcasted_iota(jnp.int32, (D, D), 1)
    ident = (di == dj).astype(jnp.float32)
    hrm_ref[...] = lax.dot_general(
        ht_ref[...], ident,
        dimension_numbers=(((0,), (0,)), ((), ())),
        preferred_element_type=jnp.float32,
    )

    scores = lax.dot_general(
        lt_ref[...], s_ref[...],
        dimension_numbers=(((0,), (0,)), ((), ())),
        preferred_element_type=jnp.float32,
    )
    rows = lax.broadcasted_iota(jnp.int32, (TILE_L, Q), 0)

    def _merge(sc):
        tmax = jnp.max(sc, axis=0, keepdims=True)
        cand = jnp.where(sc == tmax, rows, TILE_L)
        targ = jnp.min(cand, axis=0, keepdims=True) + i * TILE_L
        upd = tmax > bv_ref[...]
        bv_ref[...] = jnp.where(upd, tmax, bv_ref[...])
        bi_ref[...] = jnp.where(upd, targ, bi_ref[...])

    @pl.when(i < pl.num_programs(0) - 1)
    def _full():
        _merge(scores)

    @pl.when(i == pl.num_programs(0) - 1)
    def _tail():
        _merge(jnp.where(rows < (L - i * TILE_L), scores, -jnp.inf))
        idx_ref[...] = bi_ref[...]


def _argmax_call(lDictT, hDictT, sample):
    return pl.pallas_call(
        _argmax_body,
        grid=(N_TILES,),
        in_specs=[
            pl.BlockSpec((D, TILE_L), lambda i: (0, i)),
            pl.BlockSpec((D, TILE_L), lambda i: (0, i)),
            pl.BlockSpec((D, Q), lambda i: (0, 0)),
        ],
        out_specs=[
            pl.BlockSpec((1, Q), lambda i: (0, 0)),
            pl.BlockSpec((TILE_L, D), lambda i: (i, 0)),
        ],
        out_shape=[
            jax.ShapeDtypeStruct((1, Q), jnp.int32),
            jax.ShapeDtypeStruct((H, D), jnp.float32),
        ],
        scratch_shapes=[
            pltpu.VMEM((1, Q), jnp.float32),
            pltpu.VMEM((1, Q), jnp.int32),
        ],
    )(lDictT, hDictT, sample)


def _make_gather():
    info = plsc.get_sparse_core_info()
    nw = info.num_cores * info.num_subcores
    b_per_w = Q // nw
    mesh = plsc.VectorSubcoreMesh(core_axis_name="c", subcore_axis_name="s")

    @functools.partial(
        pl.kernel,
        mesh=mesh,
        out_type=jax.ShapeDtypeStruct((Q, D), jnp.float32),
        scratch_types=[
            pltpu.VMEM((b_per_w,), jnp.int32),
            pltpu.VMEM((b_per_w, D), jnp.float32),
            pltpu.SemaphoreType.DMA,
        ],
    )
    def gather(table_hbm, idx_hbm, out_hbm, idx_v, rows_v, sem):
        wid = lax.axis_index("s") * info.num_cores + lax.axis_index("c")
        base = wid * b_per_w
        pltpu.sync_copy(idx_hbm.at[pl.ds(base, b_per_w)], idx_v)
        handles = []
        for c in range(b_per_w // 16):
            vec = idx_v[pl.ds(c * 16, 16)]
            for j in range(16):
                handles.append(pltpu.async_copy(
                    table_hbm.at[vec[j]], rows_v.at[c * 16 + j], sem))
        for h in handles:
            h.wait()
        pltpu.sync_copy(rows_v, out_hbm.at[pl.ds(base, b_per_w)])

    return gather


@functools.lru_cache(maxsize=1)
def _gather_call():
    return _make_gather()


def kernel(sample, lDict, hDict):
    idx, hRM = _argmax_call(lDict.T, hDict.T, sample)
    return _gather_call()(hRM, idx.reshape(Q))


# speedup vs baseline: 1.5208x; 1.0681x over previous
import functools

import jax
import jax.numpy as jnp
from jax import lax
from jax.experimental import pallas as pl
from jax.experimental.pallas import tpu as pltpu
from jax.experimental.pallas import tpu_sc as plsc

L = 100000
H = 100000
D = 120
Q = 1024

TILE_L = 2048
N_TILES = pl.cdiv(L, TILE_L)


def _argmax_body(lt_ref, ht_ref, s_ref, idx_ref, hrm_ref, bv_ref, bi_ref):
    i = pl.program_id(0)

    @pl.when(i == 0)
    def _init():
        bv_ref[...] = jnp.full((1, Q), -jnp.inf, dtype=jnp.float32)
        bi_ref[...] = jnp.zeros((1, Q), dtype=jnp.int32)

    hrm_ref[...] = ht_ref[...].T

    scores = lax.dot_general(
        lt_ref[...], s_ref[...],
        dimension_numbers=(((0,), (0,)), ((), ())),
        preferred_element_type=jnp.float32,
    )
    rows = lax.broadcasted_iota(jnp.int32, (TILE_L, Q), 0)

    def _merge(sc):
        tmax = jnp.max(sc, axis=0, keepdims=True)
        cand = jnp.where(sc == tmax, rows, TILE_L)
        targ = jnp.min(cand, axis=0, keepdims=True) + i * TILE_L
        upd = tmax > bv_ref[...]
        bv_ref[...] = jnp.where(upd, tmax, bv_ref[...])
        bi_ref[...] = jnp.where(upd, targ, bi_ref[...])

    @pl.when(i < pl.num_programs(0) - 1)
    def _full():
        _merge(scores)

    @pl.when(i == pl.num_programs(0) - 1)
    def _tail():
        _merge(jnp.where(rows < (L - i * TILE_L), scores, -jnp.inf))
        idx_ref[...] = bi_ref[...]


def _argmax_call(lDictT, hDictT, sample):
    return pl.pallas_call(
        _argmax_body,
        grid=(N_TILES,),
        in_specs=[
            pl.BlockSpec((D, TILE_L), lambda i: (0, i)),
            pl.BlockSpec((D, TILE_L), lambda i: (0, i)),
            pl.BlockSpec((D, Q), lambda i: (0, 0)),
        ],
        out_specs=[
            pl.BlockSpec((1, Q), lambda i: (0, 0)),
            pl.BlockSpec((TILE_L, D), lambda i: (i, 0)),
        ],
        out_shape=[
            jax.ShapeDtypeStruct((1, Q), jnp.int32),
            jax.ShapeDtypeStruct((H, D), jnp.float32),
        ],
        scratch_shapes=[
            pltpu.VMEM((1, Q), jnp.float32),
            pltpu.VMEM((1, Q), jnp.int32),
        ],
    )(lDictT, hDictT, sample)


def _make_gather():
    info = plsc.get_sparse_core_info()
    nw = info.num_cores * info.num_subcores
    b_per_w = Q // nw
    mesh = plsc.VectorSubcoreMesh(core_axis_name="c", subcore_axis_name="s")

    @functools.partial(
        pl.kernel,
        mesh=mesh,
        out_type=jax.ShapeDtypeStruct((Q, D), jnp.float32),
        scratch_types=[
            pltpu.VMEM((b_per_w,), jnp.int32),
            pltpu.VMEM((b_per_w, D), jnp.float32),
            pltpu.SemaphoreType.DMA,
        ],
    )
    def gather(table_hbm, idx_hbm, out_hbm, idx_v, rows_v, sem):
        wid = lax.axis_index("s") * info.num_cores + lax.axis_index("c")
        base = wid * b_per_w
        pltpu.sync_copy(idx_hbm.at[pl.ds(base, b_per_w)], idx_v)
        handles = []
        for c in range(b_per_w // 16):
            vec = idx_v[pl.ds(c * 16, 16)]
            for j in range(16):
                handles.append(pltpu.async_copy(
                    table_hbm.at[vec[j]], rows_v.at[c * 16 + j], sem))
        for h in handles:
            h.wait()
        pltpu.sync_copy(rows_v, out_hbm.at[pl.ds(base, b_per_w)])

    return gather


@functools.lru_cache(maxsize=1)
def _gather_call():
    return _make_gather()


def kernel(sample, lDict, hDict):
    idx, hRM = _argmax_call(lDict.T, hDict.T, sample)
    return _gather_call()(hRM, idx.reshape(Q))
